# baseline (speedup 1.0000x reference)
import jax
import jax.numpy as jnp
from jax import lax
from jax.experimental import pallas as pl
from jax.experimental.pallas import tpu as pltpu
from jax.experimental.pallas import tpu_sc as plsc

VOCAB = 1000
VOCAB_PAD = 1024
B = 1024
SEQ = 50
SEQ_PAD = 64
ROWS_PAD = 56
N_POS = B * SEQ
NC, NS, L = 2, 16, 16
NW = NC * NS
B_PER_W = B // NW
N_CHUNKS = B_PER_W


def _prep_body(table_ref, inp_ref, tgt_ref,
               lse_ref, tpad_ref, tpad2_ref, fidxp_ref, idxp_ref):
    t = table_ref[...]
    m = jnp.max(t, axis=1)
    s = jnp.sum(jnp.exp(t - m[:, None]), axis=1)
    lse_ref[...] = m + jnp.log(s)
    zpad = jnp.zeros((VOCAB, VOCAB_PAD - VOCAB), jnp.float32)
    tpad_ref[:, :VOCAB] = t
    tpad_ref[:, VOCAB:] = zpad
    tpad2_ref[:, :VOCAB] = t
    tpad2_ref[:, VOCAB:] = zpad
    inp = inp_ref[...]
    fidxp_ref[:, :SEQ] = inp * VOCAB_PAD + tgt_ref[...]
    fidxp_ref[:, SEQ:] = jnp.zeros((B, SEQ_PAD - SEQ), jnp.int32)
    idxp_ref[:, :ROWS_PAD] = jnp.pad(inp, ((0, 0), (0, ROWS_PAD - SEQ)))


def _finish_body(parts_ref, lse_ref, table_ref, loss_ref):
    corr = (B * (SEQ_PAD - SEQ)) * (lse_ref[0] - table_ref[0, 0])
    val = (jnp.sum(parts_ref[...]) - corr) * (1.0 / N_POS)
    loss_ref[...] = val * jnp.ones((1, 1), jnp.float32)


def _sc_body(table_hbm, tflat_hbm, fidxp_hbm, idxp_hbm, lse_hbm,
             out_hbm, part_hbm,
             fidxp_v, idxp_v, rows0, rows1, tv_v, lse_g_v, acc_v,
             gsem0, gsem1, wsem0, wsem1, lsem, lsem2):
    wid = lax.axis_index("s") * NC + lax.axis_index("c")
    b0 = wid * B_PER_W
    pltpu.sync_copy(fidxp_hbm.at[pl.ds(b0, B_PER_W)], fidxp_v)
    pltpu.sync_copy(idxp_hbm.at[pl.ds(b0 * ROWS_PAD, B_PER_W * ROWS_PAD)],
                    idxp_v)

    def tv_desc(g):
        return pltpu.make_async_copy(
            tflat_hbm.at[fidxp_v.at[g]], tv_v.at[pl.ds(g * SEQ_PAD, SEQ_PAD)],
            lsem)

    def ls_desc(g):
        return pltpu.make_async_copy(
            lse_hbm.at[idxp_v.at[g]], lse_g_v.at[pl.ds(g * SEQ_PAD, SEQ_PAD)],
            lsem2)


    bufs = (rows0, rows1)
    gsems = (gsem0, gsem1)
    wsems = (wsem0, wsem1)

    def g_desc(g, b):
        return pltpu.make_async_copy(
            table_hbm.at[idxp_v.at[pl.ds(g * ROWS_PAD, ROWS_PAD)]],
            bufs[b], gsems[b])

    def w_desc(g, b):
        return pltpu.make_async_copy(
            bufs[b], out_hbm.at[pl.ds((b0 + g) * ROWS_PAD, ROWS_PAD)],
            wsems[b])

    g_desc(0, 0).start()
    g_desc(1, 1).start()
    g_desc(0, 0).wait()
    w_desc(0, 0).start()

    def pair(p, _):
        for k in (1, 2):
            g = 2 * p + k
            b = k % 2
            bp = 1 - b
            g_desc(g, b).wait()
            w_desc(g - 1, bp).wait()
            g_desc(g + 1, bp).start()
            w_desc(g, b).start()
        return 0

    lax.fori_loop(0, (N_CHUNKS - 2) // 2, pair, 0)

    last = N_CHUNKS - 1
    g_desc(last, 1).wait()
    w_desc(last - 1, 0).wait()
    w_desc(last, 1).start()
    w_desc(last, 1).wait()


    def lbody(i, acc):
        sl = pl.ds(i * L, L)
        return acc + (lse_g_v[sl] - tv_v[sl])

    acc = lax.fori_loop(0, B_PER_W * SEQ_PAD // L, lbody,
                        jnp.zeros((L,), jnp.float32))
    acc_v[...] = acc
    pltpu.sync_copy(acc_v, part_hbm.at[pl.ds(wid * L, L)])


def kernel(inputs, targets, table):
    lse, tpad, tpad2, fidxp, idxp = pl.pallas_call(
        _prep_body,
        out_shape=(
            jax.ShapeDtypeStruct((VOCAB,), jnp.float32),
            jax.ShapeDtypeStruct((VOCAB, VOCAB_PAD), jnp.float32),
            jax.ShapeDtypeStruct((VOCAB, VOCAB_PAD), jnp.float32),
            jax.ShapeDtypeStruct((B, SEQ_PAD), jnp.int32),
            jax.ShapeDtypeStruct((B, ROWS_PAD), jnp.int32),
        ),
    )(table, inputs, targets)
    table_flat = tpad2.reshape(-1)
    idxp = idxp.reshape(-1)

    mesh = plsc.VectorSubcoreMesh(core_axis_name="c", subcore_axis_name="s")
    sc = pl.kernel(
        _sc_body,
        out_type=(
            jax.ShapeDtypeStruct((B * ROWS_PAD, VOCAB), jnp.float32),
            jax.ShapeDtypeStruct((NW * L,), jnp.float32),
        ),
        mesh=mesh,
        compiler_params=pltpu.CompilerParams(use_tc_tiling_on_sc=False),
        scratch_types=[
            pltpu.VMEM((B_PER_W, SEQ_PAD), jnp.int32),
            pltpu.VMEM((B_PER_W * ROWS_PAD,), jnp.int32),
            pltpu.VMEM((ROWS_PAD, VOCAB), jnp.float32),
            pltpu.VMEM((ROWS_PAD, VOCAB), jnp.float32),
            pltpu.VMEM((B_PER_W * SEQ_PAD,), jnp.float32),
            pltpu.VMEM((B_PER_W * SEQ_PAD,), jnp.float32),
            pltpu.VMEM((L,), jnp.float32),
            pltpu.SemaphoreType.DMA,
            pltpu.SemaphoreType.DMA,
            pltpu.SemaphoreType.DMA,
            pltpu.SemaphoreType.DMA,
            pltpu.SemaphoreType.DMA,
            pltpu.SemaphoreType.DMA,
        ],
    )
    logits_pad, parts = sc(table, table_flat, fidxp, idxp, lse)

    loss = pl.pallas_call(
        _finish_body,
        out_shape=jax.ShapeDtypeStruct((1, 1), jnp.float32),
    )(parts, lse, table)[0, 0]

    return logits_pad.reshape(B, ROWS_PAD, VOCAB)[:, :SEQ, :], loss


# speedup vs baseline: 1.4380x; 1.4380x over previous
import jax
import jax.numpy as jnp
from jax import lax
from jax.experimental import pallas as pl
from jax.experimental.pallas import tpu as pltpu
from jax.experimental.pallas import tpu_sc as plsc

VOCAB = 1000
N_POS = 1024 * 50
NC, NS, L = 2, 16, 16
NW = NC * NS
PER_W = N_POS // NW
CHUNK = 40
N_CHUNKS = PER_W // CHUNK
QG = 80


def _lse_body(table_ref, lse_ref, tcopy_ref):
    t = table_ref[...]
    m = jnp.max(t, axis=1)
    s = jnp.sum(jnp.exp(t - m[:, None]), axis=1)
    lse_ref[...] = m + jnp.log(s)
    tcopy_ref[...] = t


def _finish_body(parts_ref, loss_ref):
    loss_ref[...] = jnp.sum(parts_ref[...], keepdims=True) * (1.0 / N_POS)


def _sc_body(table_hbm, tflat_hbm, idx_hbm, tgt_hbm, lse_hbm,
             out_hbm, part_hbm,
             idx_v, fidx_v, rows0, rows1, tv_v, lse_g_v, acc_v,
             gsem0, gsem1, wsem0, wsem1, lsem, lsem2):
    wid = lax.axis_index("s") * NC + lax.axis_index("c")
    base = wid * PER_W
    pltpu.sync_copy(idx_hbm.at[pl.ds(base, PER_W)], idx_v)
    pltpu.sync_copy(tgt_hbm.at[pl.ds(base, PER_W)], fidx_v)

    def fbody(i, _):
        sl = pl.ds(i * L, L)
        fidx_v[sl] = idx_v[sl] * VOCAB + fidx_v[sl]
        return 0

    lax.fori_loop(0, PER_W // L, fbody, 0)

    def tv_desc(t):
        sl = pl.ds(t * QG, QG)
        return pltpu.make_async_copy(
            tflat_hbm.at[fidx_v.at[sl]], tv_v.at[sl], lsem)

    def ls_desc(t):
        sl = pl.ds(t * QG, QG)
        return pltpu.make_async_copy(
            lse_hbm.at[idx_v.at[sl]], lse_g_v.at[sl], lsem2)

    for t in range(PER_W // QG):
        tv_desc(t).start()
        ls_desc(t).start()

    bufs = (rows0, rows1)
    gsems = (gsem0, gsem1)
    wsems = (wsem0, wsem1)

    def g_desc(g, b):
        return pltpu.make_async_copy(
            table_hbm.at[idx_v.at[pl.ds(g * CHUNK, CHUNK)]], bufs[b], gsems[b])

    def w_desc(g, b):
        return pltpu.make_async_copy(
            bufs[b], out_hbm.at[pl.ds(base + g * CHUNK, CHUNK)], wsems[b])

    g_desc(0, 0).start()
    g_desc(1, 1).start()
    g_desc(0, 0).wait()
    w_desc(0, 0).start()

    def pair(p, _):
        for k in (1, 2):
            g = 2 * p + k
            b = k % 2
            bp = 1 - b
            g_desc(g, b).wait()
            w_desc(g - 1, bp).wait()
            g_desc(g + 1, bp).start()
            w_desc(g, b).start()
        return 0

    lax.fori_loop(0, (N_CHUNKS - 2) // 2, pair, 0)

    last = N_CHUNKS - 1
    g_desc(last, 1).wait()
    w_desc(last - 1, 0).wait()
    w_desc(last, 1).start()
    w_desc(last, 1).wait()

    for t in range(PER_W // QG):
        tv_desc(t).wait()
        ls_desc(t).wait()

    def lbody(i, acc):
        sl = pl.ds(i * L, L)
        return acc + (lse_g_v[sl] - tv_v[sl])

    acc = lax.fori_loop(0, PER_W // L, lbody, jnp.zeros((L,), jnp.float32))
    acc_v[...] = acc
    pltpu.sync_copy(acc_v, part_hbm.at[wid])


def kernel(inputs, targets, table):
    B, Ln = inputs.shape
    idx_flat = inputs.reshape(-1)
    tgt_flat = targets.reshape(-1)

    lse, tcopy = pl.pallas_call(
        _lse_body,
        out_shape=(
            jax.ShapeDtypeStruct((VOCAB,), jnp.float32),
            jax.ShapeDtypeStruct((VOCAB, VOCAB), jnp.float32),
        ),
    )(table)
    table_flat = tcopy.reshape(-1)

    mesh = plsc.VectorSubcoreMesh(core_axis_name="c", subcore_axis_name="s")
    sc = pl.kernel(
        _sc_body,
        out_type=(
            jax.ShapeDtypeStruct((N_POS, VOCAB), jnp.float32),
            jax.ShapeDtypeStruct((NW, L), jnp.float32),
        ),
        mesh=mesh,
        compiler_params=pltpu.CompilerParams(use_tc_tiling_on_sc=False),
        scratch_types=[
            pltpu.VMEM((PER_W,), jnp.int32),
            pltpu.VMEM((PER_W,), jnp.int32),
            pltpu.VMEM((CHUNK, VOCAB), jnp.float32),
            pltpu.VMEM((CHUNK, VOCAB), jnp.float32),
            pltpu.VMEM((PER_W,), jnp.float32),
            pltpu.VMEM((PER_W,), jnp.float32),
            pltpu.VMEM((L,), jnp.float32),
            pltpu.SemaphoreType.DMA,
            pltpu.SemaphoreType.DMA,
            pltpu.SemaphoreType.DMA,
            pltpu.SemaphoreType.DMA,
            pltpu.SemaphoreType.DMA,
            pltpu.SemaphoreType.DMA,
        ],
    )
    logits_flat, parts = sc(table, table_flat, idx_flat, tgt_flat, lse)

    loss = pl.pallas_call(
        _finish_body,
        out_shape=jax.ShapeDtypeStruct((1, 1), jnp.float32),
    )(parts)[0, 0]

    return logits_flat.reshape(B, Ln, VOCAB), loss
